# Initial kernel scaffold; baseline (speedup 1.0000x reference)
import jax
import jax.numpy as jnp
from jax.experimental import pallas as pl


def kernel(xyz, params):
    raise NotImplementedError("write your pallas kernel here")


# speedup vs baseline: 2.5699x; 2.5699x over previous
import functools

import jax
import jax.numpy as jnp
import numpy as np
from jax.experimental import pallas as pl
from jax.experimental.pallas import tpu as pltpu

BS = 4
N = 2048
RATE = 0.9
N1 = int(N * RATE)
N2 = int(N * RATE ** 2)
N3 = int(N * RATE ** 3)

_INF = 3.0e38


def _rup(x, m):
    return (x + m - 1) // m * m


def _fps_body(xyz_ref, out_ref, *, npoint, mreal):
    xs = xyz_ref[0]
    ys = xyz_ref[1]
    zs = xyz_ref[2]
    b, mp = xs.shape
    np_pad = out_ref.shape[1]
    lane = jax.lax.broadcasted_iota(jnp.int32, (b, mp), 1)
    oiota = jax.lax.broadcasted_iota(jnp.int32, (b, np_pad), 1)
    riota = jax.lax.broadcasted_iota(jnp.int32, (b, np_pad), 0)
    oiota = oiota + jnp.minimum(riota, 0)
    dist0 = jnp.where(lane < mreal, jnp.float32(1e10), jnp.float32(-1.0))
    far0 = jnp.zeros((b, 1), jnp.int32)
    cent0 = jnp.zeros((b, np_pad), jnp.int32)

    def body(i, carry):
        distance, far, cent = carry
        cent = cent + (oiota == i).astype(jnp.int32) * far
        onehot = lane == far
        cx = jnp.sum(jnp.where(onehot, xs, 0.0), axis=1, keepdims=True)
        cy = jnp.sum(jnp.where(onehot, ys, 0.0), axis=1, keepdims=True)
        cz = jnp.sum(jnp.where(onehot, zs, 0.0), axis=1, keepdims=True)
        dx = xs - cx
        dy = ys - cy
        dz = zs - cz
        d = dx * dx + dy * dy + dz * dz
        distance = jnp.minimum(distance, d)
        m = jnp.max(distance, axis=1, keepdims=True)
        far = jnp.min(jnp.where(distance == m, lane, mp), axis=1, keepdims=True)
        return distance, far.astype(jnp.int32), cent

    _, _, cent = jax.lax.fori_loop(0, npoint, body, (dist0, far0, cent0))
    out_ref[...] = cent


def _fps(xyz, npoint, mreal):
    b, m, _ = xyz.shape
    bp = _rup(b, 8)
    mp = _rup(m, 128)
    xt = jnp.transpose(xyz, (2, 0, 1))
    xt = jnp.pad(xt, ((0, 0), (0, bp - b), (0, mp - m)))
    np_pad = _rup(npoint, 128)
    out = pl.pallas_call(
        functools.partial(_fps_body, npoint=npoint, mreal=mreal),
        out_shape=jax.ShapeDtypeStruct((bp, np_pad), jnp.int32),
    )(xt)
    return out[:b, :npoint]


def _knn_body(pcm_ref, ppm_ref, out_ref, *, k, mreal, mp, bq):
    a = pcm_ref[0]
    xs = a[0:1, :]
    ys = a[1:2, :]
    zs = a[2:3, :]
    q = ppm_ref[0]
    xq = q[:, 0:1]
    yq = q[:, 1:2]
    zq = q[:, 2:3]
    p2 = xs * xs + ys * ys + zs * zs
    q2 = xq * xq + yq * yq + zq * zq
    dot = jax.lax.dot_general(q, a, (((1,), (0,)), ((), ())),
                              preferred_element_type=jnp.float32)
    d = (q2 + p2) - 2.0 * dot
    lane = jax.lax.broadcasted_iota(jnp.int32, (bq, mp), 1)
    d = jnp.where(lane < mreal, d, _INF)
    kiota = jax.lax.broadcasted_iota(jnp.int32, (bq, k), 1)
    idxs0 = jnp.zeros((bq, k), jnp.int32)

    def body(i, carry):
        d, idxs = carry
        m = jnp.min(d, axis=1, keepdims=True)
        eq = d == m
        idx = jnp.min(jnp.where(eq, lane, mp), axis=1, keepdims=True)
        idxs = idxs + (kiota == i).astype(jnp.int32) * idx
        d = jnp.where(lane == idx, _INF, d)
        return d, idxs

    _, idxs = jax.lax.fori_loop(0, k, body, (d, idxs0))
    out_ref[0] = idxs


def _knn(xyz, k, mreal):
    b, m, _ = xyz.shape
    mp = _rup(m, 128)
    bq = 256
    nqb = mp // bq
    pcm = jnp.pad(jnp.transpose(xyz, (2, 0, 1)), ((0, 0), (0, 0), (0, mp - m)))
    pcm = jnp.transpose(pcm, (1, 0, 2))
    ppm = jnp.pad(xyz, ((0, 0), (0, mp - m), (0, 0)))
    out = pl.pallas_call(
        functools.partial(_knn_body, k=k, mreal=mreal, mp=mp, bq=bq),
        grid=(b, nqb),
        in_specs=[
            pl.BlockSpec((1, 3, mp), lambda i, j: (i, 0, 0)),
            pl.BlockSpec((1, bq, 3), lambda i, j: (i, j, 0)),
        ],
        out_specs=pl.BlockSpec((1, bq, k), lambda i, j: (i, j, 0)),
        out_shape=jax.ShapeDtypeStruct((b, mp, k), jnp.int32),
    )(pcm, ppm)
    return out[:, :m, :]


def _index_points(points, idx):
    b = points.shape[0]
    bidx = jnp.arange(b).reshape((b,) + (1,) * (idx.ndim - 1))
    return points[bidx, idx]


def _apply_mlp(layers, x, final_proc, conv_dim):
    axes = (0,) + tuple(range(2, 2 + conv_dim))
    nl = len(layers)
    for i, lyr in enumerate(layers):
        if conv_dim == 1:
            x = jnp.einsum('oc,bcn->bon', lyr['W'], x) + lyr['b'][None, :, None]
        else:
            x = jnp.einsum('oc,bcnk->bonk', lyr['W'], x) + lyr['b'][None, :, None, None]
        if i < nl - 1 or final_proc:
            mean = jnp.mean(x, axis=axes, keepdims=True)
            var = jnp.var(x, axis=axes, keepdims=True)
            xn = (x - mean) / jnp.sqrt(var + 1e-5)
            g = lyr['gamma'].reshape((1, -1) + (1,) * conv_dim)
            bta = lyr['beta'].reshape((1, -1) + (1,) * conv_dim)
            x = jax.nn.relu(xn * g + bta)
    return x


def _square_distance(src, dst):
    return (jnp.sum(src ** 2, -1)[:, :, None] + jnp.sum(dst ** 2, -1)[:, None, :]
            - 2.0 * jnp.einsum('bnc,bmc->bnm', src, dst))


def _down_sample(xyz, fea, n_center, n_near, layers):
    m = xyz.shape[1]
    knn_idx = _knn(xyz, n_near, m)
    fps_idx = _fps(xyz, n_center, m)
    center_xyz = _index_points(xyz, fps_idx)
    group_idx = _index_points(knn_idx, fps_idx)
    group_xyz = _index_points(xyz, group_idx)
    xyz_rel = group_xyz - center_xyz[:, :, None, :]
    group_fea = _index_points(fea, group_idx)
    group_fea = jnp.concatenate([group_fea, xyz_rel], -1).transpose(0, 3, 1, 2)
    new_fea = _apply_mlp(layers, group_fea, True, 2)
    new_fea = jnp.max(new_fea, axis=3).transpose(0, 2, 1)
    return center_xyz, new_fea


def _up_sample(xyz1, xyz2, fea1, fea2, layers):
    d = _square_distance(xyz1, xyz2)
    negd, idx = jax.lax.top_k(-d, 3)
    dists = -negd
    dist_recip = 1.0 / (dists + 1e-08)
    norm = jnp.sum(dist_recip, 2, keepdims=True)
    weight = dist_recip / norm
    interp = jnp.sum(_index_points(fea2, idx) * weight[:, :, :, None], axis=2)
    new_fea = jnp.concatenate([fea1, interp], -1)
    return _apply_mlp(layers, new_fea.transpose(0, 2, 1), False, 1).transpose(0, 2, 1)


def kernel(xyz, params):
    l1_xyz, l1_fea = _down_sample(xyz, xyz, N1, 50, params['dn1'])
    l2_xyz, l2_fea = _down_sample(l1_xyz, l1_fea, N2, 40, params['dn2'])
    l3_xyz, l3_fea = _down_sample(l2_xyz, l2_fea, N3, 30, params['dn3'])
    l2_fea = _up_sample(l2_xyz, l3_xyz, l2_fea, l3_fea, params['up3'])
    l1_fea = _up_sample(l1_xyz, l2_xyz, l1_fea, l2_fea, params['up2'])
    l0_fea = _up_sample(xyz, l1_xyz, jnp.concatenate([xyz, xyz], 2), l1_fea,
                        params['up1']).transpose(0, 2, 1)
    pmt = _apply_mlp(params['pmt'], l0_fea, False, 1).transpose(0, 2, 1)
    log_pmt = jax.nn.log_softmax(pmt, axis=2)
    mad = _apply_mlp(params['mad'], l0_fea, False, 1).transpose(0, 2, 1)
    dim = jnp.squeeze(_apply_mlp(params['dim'], l0_fea, False, 1))
    nor = _apply_mlp(params['nor'], l0_fea, False, 1).transpose(0, 2, 1)
    loc = _apply_mlp(params['loc'], l0_fea, False, 1).transpose(0, 2, 1)
    return (log_pmt, mad, dim, nor, loc)
